# baseline (speedup 1.0000x reference)
import functools
import math

import jax
import jax.numpy as jnp
import numpy as np
from jax import lax
from jax.experimental import pallas as pl
from jax.experimental.pallas import tpu as pltpu
from jax.experimental.pallas import tpu_sc as plsc

NC = 2
NS = 16
NW = NC * NS
L = 16

C = 8


@functools.lru_cache(maxsize=None)
def _pe_tables_np(T: int, d_model: int):
    M = T // C
    w = np.exp(np.arange(0, d_model, 2, dtype=np.float64)
               * (-math.log(10000.0) / d_model))
    a = (C * np.arange(M, dtype=np.float64))[:, None] * w[None, :]
    b = np.arange(C, dtype=np.float64)[:, None] * w[None, :]

    A = np.empty((M, d_model), dtype=np.float32)
    A[:, 0::2] = np.sin(a)
    A[:, 1::2] = np.cos(a)
    Asw = np.empty((M, d_model), dtype=np.float32)
    Asw[:, 0::2] = np.cos(a)
    Asw[:, 1::2] = np.sin(a)
    U = np.empty((C, d_model), dtype=np.float32)
    U[:, 0::2] = np.cos(b)
    U[:, 1::2] = np.cos(b)
    V = np.empty((C, d_model), dtype=np.float32)
    V[:, 0::2] = np.sin(b)
    V[:, 1::2] = -np.sin(b)
    return A, Asw, U, V


@functools.lru_cache(maxsize=None)
def _build_sc_kernel(B: int, T: int, D: int, n_chunks: int):
    t_per_w = T // NW
    vregs_per_row = D // L

    mesh = plsc.VectorSubcoreMesh(core_axis_name="c", subcore_axis_name="s")

    @functools.partial(
        pl.kernel,
        out_type=jax.ShapeDtypeStruct((B, T, D), jnp.float32),
        mesh=mesh,
        scratch_types=[
            pltpu.VMEM((B, t_per_w), jnp.int32),
            pltpu.VMEM((n_chunks, D), jnp.float32),
            pltpu.VMEM((n_chunks, D), jnp.float32),
            pltpu.VMEM((C, D), jnp.float32),
            pltpu.VMEM((C, D), jnp.float32),
            [pltpu.VMEM((B * C, D), jnp.float32) for _ in range(2)],
            [pltpu.SemaphoreType.DMA for _ in range(2)],
            [pltpu.SemaphoreType.DMA for _ in range(2)],
        ],
    )
    def k(ids_hbm, a_hbm, asw_hbm, u_hbm, v_hbm, table_hbm,
          out_hbm, ids_v, a_v, asw_v, u_v, v_v, rows_v, gsem, ssem):
        sid = lax.axis_index("s")
        w = sid * NC + lax.axis_index("c")
        t0 = w * t_per_w
        m0 = w * n_chunks

        for b in range(B):
            pltpu.sync_copy(ids_hbm.at[b, pl.ds(t0, t_per_w)], ids_v.at[b])
        pltpu.sync_copy(a_hbm.at[pl.ds(m0, n_chunks)], a_v)
        pltpu.sync_copy(asw_hbm.at[pl.ds(m0, n_chunks)], asw_v)
        pltpu.sync_copy(u_hbm, u_v)
        pltpu.sync_copy(v_hbm, v_v)

        gh = [None, None]
        sh = [None, None]

        def start(c):
            p = c % 2
            gh[p] = [
                pltpu.async_copy(
                    table_hbm.at[ids_v.at[b, pl.ds(c * C, C)]],
                    rows_v[p].at[pl.ds(b * C, C)],
                    gsem[p],
                )
                for b in range(B)
            ]

        def apply_pe(c, p):
            def pe_body(v, _, c=c, p=p):
                lane = pl.ds(v * L, L)
                a_vec = a_v[c, lane]
                asw_vec = asw_v[c, lane]
                for j in range(C):
                    pe = a_vec * u_v[j, lane] + asw_vec * v_v[j, lane]
                    for b in range(B):
                        plsc.addupdate(rows_v[p].at[b * C + j, lane], pe)
                return 0

            lax.fori_loop(0, vregs_per_row, pe_body, 0, unroll=2)

        start(0)
        for c in range(n_chunks):
            p = c % 2
            if c + 1 < n_chunks:
                if sh[1 - p] is not None:
                    for h in sh[1 - p]:
                        h.wait()
                    sh[1 - p] = None
                start(c + 1)
            for h in gh[p]:
                h.wait()
            apply_pe(c, p)
            tc0 = t0 + c * C
            sh[p] = [
                pltpu.async_copy(
                    rows_v[p].at[pl.ds(b * C, C)],
                    out_hbm.at[b, pl.ds(tc0, C)],
                    ssem[p],
                )
                for b in range(B)
            ]
        for p in range(2):
            if sh[p] is not None:
                for h in sh[p]:
                    h.wait()

    return k


def kernel(token_ids, token_emb_weight):
    B, T = token_ids.shape
    V, D = token_emb_weight.shape
    t_per_w = T // NW
    n_chunks = t_per_w // C

    ids = token_ids.astype(jnp.int32)
    A, Asw, U, Vt = (jnp.asarray(x) for x in _pe_tables_np(T, D))
    k = _build_sc_kernel(B, T, D, n_chunks)
    return k(ids, A, Asw, U, Vt, token_emb_weight)


# speedup vs baseline: 1.3652x; 1.3652x over previous
import functools
import math

import jax
import jax.numpy as jnp
import numpy as np
from jax import lax
from jax.experimental import pallas as pl
from jax.experimental.pallas import tpu as pltpu
from jax.experimental.pallas import tpu_sc as plsc

NC = 2
NS = 16
NW = NC * NS
L = 16

C = 8


@functools.lru_cache(maxsize=None)
def _pe_tables_np(T: int, d_model: int):
    M = T // C
    w = np.exp(np.arange(0, d_model, 2, dtype=np.float64)
               * (-math.log(10000.0) / d_model))
    a = (C * np.arange(M, dtype=np.float64))[:, None] * w[None, :]
    b = np.arange(C, dtype=np.float64)[:, None] * w[None, :]

    A = np.empty((M, d_model), dtype=np.float32)
    A[:, 0::2] = np.sin(a)
    A[:, 1::2] = np.cos(a)
    Asw = np.empty((M, d_model), dtype=np.float32)
    Asw[:, 0::2] = np.cos(a)
    Asw[:, 1::2] = np.sin(a)
    U = np.empty((C, d_model), dtype=np.float32)
    U[:, 0::2] = np.cos(b)
    U[:, 1::2] = np.cos(b)
    V = np.empty((C, d_model), dtype=np.float32)
    V[:, 0::2] = np.sin(b)
    V[:, 1::2] = -np.sin(b)
    return A, Asw, U, V


@functools.lru_cache(maxsize=None)
def _build_sc_kernel(B: int, T: int, D: int, n_chunks: int):
    t_per_w = T // NW
    vregs_per_row = D // L

    mesh = plsc.VectorSubcoreMesh(core_axis_name="c", subcore_axis_name="s")

    @functools.partial(
        pl.kernel,
        out_type=jax.ShapeDtypeStruct((B, T, D), jnp.float32),
        mesh=mesh,
        scratch_types=[
            pltpu.VMEM((B, t_per_w), jnp.int32),
            pltpu.VMEM((n_chunks, D), jnp.float32),
            pltpu.VMEM((n_chunks, D), jnp.float32),
            pltpu.VMEM((C, D), jnp.float32),
            pltpu.VMEM((C, D), jnp.float32),
            [pltpu.VMEM((B * C, D), jnp.float32) for _ in range(2)],
            [pltpu.SemaphoreType.DMA for _ in range(2)],
            [pltpu.SemaphoreType.DMA for _ in range(2)],
        ],
    )
    def k(ids_hbm, a_hbm, asw_hbm, u_hbm, v_hbm, table_hbm,
          out_hbm, ids_v, a_v, asw_v, u_v, v_v, rows_v, gsem, ssem):
        sid = lax.axis_index("s")
        w = sid * NC + lax.axis_index("c")
        t0 = w * t_per_w
        m0 = w * n_chunks

        for b in range(B):
            pltpu.sync_copy(ids_hbm.at[b, pl.ds(t0, t_per_w)], ids_v.at[b])
        pltpu.sync_copy(a_hbm.at[pl.ds(m0, n_chunks)], a_v)
        pltpu.sync_copy(asw_hbm.at[pl.ds(m0, n_chunks)], asw_v)
        pltpu.sync_copy(u_hbm, u_v)
        pltpu.sync_copy(v_hbm, v_v)

        gh = [None, None]
        sh = [None, None]

        def start(c):
            p = c % 2
            gh[p] = [
                pltpu.async_copy(
                    table_hbm.at[ids_v.at[b, pl.ds(c * C, C)]],
                    rows_v[p].at[pl.ds(b * C, C)],
                    gsem[p],
                )
                for b in range(B)
            ]

        def apply_pe(c, p):
            def pe_body(v, _, c=c, p=p):
                lane = pl.ds(v * L, L)
                a_vec = a_v[c, lane]
                asw_vec = asw_v[c, lane]
                for j in range(C):
                    pe = a_vec * u_v[j, lane] + asw_vec * v_v[j, lane]
                    for b in range(B):
                        plsc.addupdate(rows_v[p].at[b * C + j, lane], pe)
                return 0

            lax.fori_loop(0, vregs_per_row, pe_body, 0, unroll=8)

        start(0)
        for c in range(n_chunks):
            p = c % 2
            if c + 1 < n_chunks:
                if sh[1 - p] is not None:
                    for h in sh[1 - p]:
                        h.wait()
                    sh[1 - p] = None
                start(c + 1)
            for h in gh[p]:
                h.wait()
            apply_pe(c, p)
            tc0 = t0 + c * C
            sh[p] = [
                pltpu.async_copy(
                    rows_v[p].at[pl.ds(b * C, C)],
                    out_hbm.at[b, pl.ds(tc0, C)],
                    ssem[p],
                )
                for b in range(B)
            ]
        for p in range(2):
            if sh[p] is not None:
                for h in sh[p]:
                    h.wait()

    return k


def kernel(token_ids, token_emb_weight):
    B, T = token_ids.shape
    V, D = token_emb_weight.shape
    t_per_w = T // NW
    n_chunks = t_per_w // C

    ids = token_ids.astype(jnp.int32)
    A, Asw, U, Vt = (jnp.asarray(x) for x in _pe_tables_np(T, D))
    k = _build_sc_kernel(B, T, D, n_chunks)
    return k(ids, A, Asw, U, Vt, token_emb_weight)
